# Initial kernel scaffold; baseline (speedup 1.0000x reference)
import jax
import jax.numpy as jnp
from jax.experimental import pallas as pl


def kernel(features, edge_index, edge_types, W_et, b_et, W_ih, W_hh, b_ih, b_hh, W_cls, b_cls):
    raise NotImplementedError("write your pallas kernel here")


# speedup vs baseline: 24.9844x; 24.9844x over previous
import functools

import jax
import jax.numpy as jnp
from jax import lax
from jax.experimental import pallas as pl
from jax.experimental.pallas import tpu as pltpu
from jax.experimental.pallas import tpu_sc as plsc

N = 10000
E = 320000
D = 128
T = 4
STEPS = 8
B = 10
NPG = 1000

NC = 2
NS = 16
NW = NC * NS
EPW = E // NW
K = 80
NCH = EPW // K
NP_ = 10240
RPS = NP_ // NS

@functools.lru_cache(maxsize=None)
def _get_sc_aggregate():
    mesh = plsc.VectorSubcoreMesh(core_axis_name="c", subcore_axis_name="s")

    @functools.partial(
        pl.kernel,
        mesh=mesh,
        out_type=jax.ShapeDtypeStruct((NC, NP_, D), jnp.float32),
        scratch_types=[
            pltpu.VMEM((NCH, K), jnp.int32),
            pltpu.VMEM((NCH, K), jnp.int32),
            pltpu.VMEM((K, D), jnp.float32),
            pltpu.VMEM_SHARED((NP_, D), jnp.float32),
            pltpu.SemaphoreType.DMA,
        ],
    )
    def _sc_aggregate(table, gidx, dst, zeros, out, gidx_v, dst_v, rows_v, accum, sem):
        s = lax.axis_index("s")
        c = lax.axis_index("c")
        wid = s * NC + c

        pltpu.sync_copy(gidx.at[wid], gidx_v)
        pltpu.sync_copy(dst.at[wid], dst_v)
        pltpu.sync_copy(zeros.at[pl.ds(s * RPS, RPS)], accum.at[pl.ds(s * RPS, RPS)])
        plsc.subcore_barrier()

        def body(j, carry):
            pltpu.async_copy(table.at[gidx_v.at[j]], rows_v, sem).wait()
            pltpu.sync_copy(rows_v, accum.at[dst_v.at[j]], add=True)
            return carry

        lax.fori_loop(0, NCH, body, 0)

        plsc.subcore_barrier()
        pltpu.sync_copy(accum.at[pl.ds(s * RPS, RPS)],
                        out.at[c, pl.ds(s * RPS, RPS)])

    return _sc_aggregate


_BN = 2000


def _transform_body(h_ref, w_ref, b_ref, out_ref):
    h = h_ref[...]
    w = w_ref[0]
    out_ref[0] = lax.dot_general(h, w, (((1,), (1,)), ((), ())),
                                 preferred_element_type=jnp.float32) + b_ref[0]


def _tc_transform(h, W_et, b_et):
    return pl.pallas_call(
        _transform_body,
        grid=(N // _BN, T),
        in_specs=[
            pl.BlockSpec((_BN, D), lambda i, t: (i, 0)),
            pl.BlockSpec((1, D, D), lambda i, t: (t, 0, 0)),
            pl.BlockSpec((1, 1, D), lambda i, t: (t, 0, 0)),
        ],
        out_specs=pl.BlockSpec((1, _BN, D), lambda i, t: (t, i, 0)),
        out_shape=jax.ShapeDtypeStruct((T, N, D), jnp.float32),
    )(h, W_et, b_et)


def _gru_body(p0_ref, p1_ref, h_ref, wih_ref, whh_ref, bih_ref, bhh_ref, out_ref):
    a = p0_ref[...] + p1_ref[...]
    h = h_ref[...]
    gi = lax.dot_general(a, wih_ref[...], (((1,), (1,)), ((), ())),
                         preferred_element_type=jnp.float32) + bih_ref[...]
    gh = lax.dot_general(h, whh_ref[...], (((1,), (1,)), ((), ())),
                         preferred_element_type=jnp.float32) + bhh_ref[...]
    r = jax.nn.sigmoid(gi[:, :D] + gh[:, :D])
    z = jax.nn.sigmoid(gi[:, D:2 * D] + gh[:, D:2 * D])
    n = jnp.tanh(gi[:, 2 * D:] + r * gh[:, 2 * D:])
    out_ref[...] = (1.0 - z) * n + z * h


def _tc_gru(p0, p1, h, W_ih, W_hh, b_ih, b_hh):
    return pl.pallas_call(
        _gru_body,
        grid=(N // _BN,),
        in_specs=[
            pl.BlockSpec((_BN, D), lambda i: (i, 0)),
            pl.BlockSpec((_BN, D), lambda i: (i, 0)),
            pl.BlockSpec((_BN, D), lambda i: (i, 0)),
            pl.BlockSpec((3 * D, D), lambda i: (0, 0)),
            pl.BlockSpec((3 * D, D), lambda i: (0, 0)),
            pl.BlockSpec((1, 3 * D), lambda i: (0, 0)),
            pl.BlockSpec((1, 3 * D), lambda i: (0, 0)),
        ],
        out_specs=pl.BlockSpec((_BN, D), lambda i: (i, 0)),
        out_shape=jax.ShapeDtypeStruct((N, D), jnp.float32),
    )(p0, p1, h, W_ih, W_hh, b_ih, b_hh)


def _pool_body(h_ref, w_ref, b_ref, out_ref):
    s = jnp.sum(h_ref[...], axis=0, keepdims=True)
    logit = jnp.sum(s * w_ref[...], axis=1, keepdims=True)
    out_ref[0] = jax.nn.sigmoid(logit + b_ref[...])


def _tc_pool(h, W_cls, b_cls):
    return pl.pallas_call(
        _pool_body,
        grid=(B,),
        in_specs=[
            pl.BlockSpec((NPG, D), lambda i: (i, 0)),
            pl.BlockSpec((1, D), lambda i: (0, 0)),
            pl.BlockSpec((1, D), lambda i: (0, 0)),
        ],
        out_specs=pl.BlockSpec((1, 1, D), lambda i: (i, 0, 0)),
        out_shape=jax.ShapeDtypeStruct((B, 1, D), jnp.float32),
    )(h, W_cls, b_cls)


def kernel(features, edge_index, edge_types, W_et, b_et, W_ih, W_hh, b_ih, b_hh, W_cls, b_cls):
    src = edge_index[0]
    dst = edge_index[1]
    gidx = (edge_types * N + src).reshape(NW, NCH, K)
    dst3 = dst.reshape(NW, NCH, K)
    zeros = jnp.zeros((NP_, D), jnp.float32)
    b_ih2 = b_ih.reshape(1, 3 * D)
    b_hh2 = b_hh.reshape(1, 3 * D)
    b_cls2 = jnp.broadcast_to(b_cls.reshape(1, 1), (1, D))

    h = features
    for _ in range(STEPS):
        table = _tc_transform(h, W_et, b_et.reshape(T, 1, D)).reshape(T * N, D)
        parts = _get_sc_aggregate()(table, gidx, dst3, zeros)
        h = _tc_gru(parts[0, :N], parts[1, :N], h, W_ih, W_hh, b_ih2, b_hh2)
    out = _tc_pool(h, W_cls, b_cls2)
    return out[:, 0, 0]
